# baseline (speedup 1.0000x reference)
import jax
import jax.numpy as jnp
from jax import lax
from jax.experimental import pallas as pl
from jax.experimental.pallas import tpu as pltpu
from jax.experimental.pallas import tpu_sc as plsc

N_ROWS = 1_000_000
DIM = 64
BATCH = 819_200
NPAIR = N_ROWS // 2

NUM_CORES = 2
NUM_SUBCORES = 16
LANES = 16
NUM_WORKERS = NUM_CORES * NUM_SUBCORES

C1_ROWS = 400
C1_CHUNKS = N_ROWS // C1_ROWS
C1_PER_W = -(-C1_CHUNKS // NUM_WORKERS)

B_PER_W = BATCH // NUM_WORKERS
C2 = 128
SUB = 128
KSUB = C2 // SUB
C2_CHUNKS = B_PER_W // C2


def _compact_body(table_hbm, pairs_hbm, a_v, b_v):
    wid = lax.axis_index("s") * NUM_CORES + lax.axis_index("c")

    def chunk(i, carry):
        cid = wid + i * NUM_WORKERS

        @pl.when(cid < C1_CHUNKS)
        def _():
            r0 = pl.multiple_of(cid * C1_ROWS, C1_ROWS)
            pltpu.sync_copy(table_hbm.at[pl.ds(r0, C1_ROWS), :], a_v)

            def pair(p, c):
                for cc in range(DIM // LANES):
                    b_v[p, pl.ds(cc * LANES, LANES)] = (
                        a_v[2 * p, pl.ds(cc * LANES, LANES)])
                    b_v[p, pl.ds(DIM + cc * LANES, LANES)] = (
                        a_v[2 * p + 1, pl.ds(cc * LANES, LANES)])
                return c
            lax.fori_loop(0, C1_ROWS // 2, pair, 0)
            p0 = pl.multiple_of(r0 // 2, C1_ROWS // 2)
            pltpu.sync_copy(b_v, pairs_hbm.at[pl.ds(p0, C1_ROWS // 2), :])
        return carry

    lax.fori_loop(0, C1_PER_W, chunk, 0)


def _sample_body(pairs_hbm, param_hbm, out_hbm,
                 param_v, w_v, s0_v, s1_v, idxa_v, idxb_v,
                 bufa_v, bufb_v, res_v, sem):
    wid = lax.axis_index("s") * NUM_CORES + lax.axis_index("c")
    base = wid * B_PER_W
    scale = jnp.float32(N_ROWS - 1)

    def chunk(g, carry):
        off = pl.multiple_of(base + g * C2, C2)
        pltpu.sync_copy(param_hbm.at[pl.ds(off, C2)], param_v)

        for j in range(C2 // LANES):
            p = param_v[pl.ds(j * LANES, LANES)]
            t = jnp.minimum(jnp.maximum(p, 0.0), 1.0) * scale
            i0 = t.astype(jnp.int32)
            i1 = jnp.minimum(i0 + 1, N_ROWS - 1)
            w = t - i0.astype(jnp.float32)
            k, r = divmod(j * LANES, SUB)
            idxa_v[k, pl.ds(r, LANES)] = lax.shift_right_logical(i0, 1)
            idxb_v[k, pl.ds(r, LANES)] = lax.shift_right_logical(i1, 1)
            s0_v[pl.ds(j * LANES, LANES)] = lax.shift_left(
                jnp.bitwise_and(i0, 1), 6)
            s1_v[pl.ds(j * LANES, LANES)] = lax.shift_left(
                jnp.bitwise_and(i1, 1), 6)
            w_v[pl.ds(j * LANES, LANES)] = w

        copies = []
        for k in range(KSUB):
            copies.append(
                pltpu.async_copy(pairs_hbm.at[idxa_v.at[k]], bufa_v.at[k], sem))
            copies.append(
                pltpu.async_copy(pairs_hbm.at[idxb_v.at[k]], bufb_v.at[k], sem))
        for cp in copies:
            cp.wait()

        for k in range(KSUB):
            def row16(r16, c, _k=k):
                rr = _k * SUB + r16 * LANES
                w16 = w_v[pl.ds(rr, LANES)]
                s0_16 = s0_v[pl.ds(rr, LANES)]
                s1_16 = s1_v[pl.ds(rr, LANES)]
                for j in range(LANES):
                    r = r16 * LANES + j
                    wb = w16.at[jnp.full((LANES,), j, jnp.int32)].get(
                        mode="promise_in_bounds")
                    one_m = 1.0 - wb
                    s0 = s0_16[j]
                    s1 = s1_16[j]
                    for cc in range(DIM // LANES):
                        v0 = bufa_v[_k, r, pl.ds(s0 + cc * LANES, LANES)]
                        v1 = bufb_v[_k, r, pl.ds(s1 + cc * LANES, LANES)]
                        res_v[_k * SUB + r, pl.ds(cc * LANES, LANES)] = (
                            v0 * one_m + v1 * wb)
                return c
            lax.fori_loop(0, SUB // LANES, row16, 0)

        pltpu.sync_copy(res_v, out_hbm.at[pl.ds(off, C2), :])
        return carry

    lax.fori_loop(0, C2_CHUNKS, chunk, 0)


@jax.jit
def kernel(input, param):
    mesh = plsc.VectorSubcoreMesh(core_axis_name="c", subcore_axis_name="s")
    params = pltpu.CompilerParams(use_tc_tiling_on_sc=True)

    pairs = pl.kernel(
        _compact_body,
        out_type=jax.ShapeDtypeStruct((NPAIR, 2 * DIM), jnp.float32),
        mesh=mesh,
        scratch_types=[
            pltpu.VMEM((C1_ROWS, DIM), jnp.float32),
            pltpu.VMEM((C1_ROWS // 2, 2 * DIM), jnp.float32),
        ],
        compiler_params=params,
    )(input)

    out = pl.kernel(
        _sample_body,
        out_type=jax.ShapeDtypeStruct((BATCH, DIM), jnp.float32),
        mesh=mesh,
        scratch_types=[
            pltpu.VMEM((C2,), jnp.float32),
            pltpu.VMEM((C2,), jnp.float32),
            pltpu.VMEM((C2,), jnp.int32),
            pltpu.VMEM((C2,), jnp.int32),
            pltpu.VMEM((KSUB, SUB), jnp.int32),
            pltpu.VMEM((KSUB, SUB), jnp.int32),
            pltpu.VMEM((KSUB, SUB, 2 * DIM), jnp.float32),
            pltpu.VMEM((KSUB, SUB, 2 * DIM), jnp.float32),
            pltpu.VMEM((C2, DIM), jnp.float32),
            pltpu.SemaphoreType.DMA,
        ],
        compiler_params=params,
    )(pairs, param)
    return out
